# baseline (speedup 1.0000x reference)
import functools

import jax
import jax.numpy as jnp
from jax import lax
from jax.experimental import pallas as pl
from jax.experimental.pallas import tpu as pltpu
from jax.experimental.pallas import tpu_sc as plsc

N_NODES = 10000
D = 128
EPS = 1e-5

NC = 2
NS = 16
NW = NC * NS
C = 128
NCHUNK = 80
EPT = C * NCHUNK
E_PAD = NW * EPT
AGG_ROWS = 10240
DUMMY_DST = 10100
ZROWS_PER_TILE = AGG_ROWS // NS
OROWS_PER_TILE = N_NODES // NS


def _lin_body(x_ref, wt_ref, b_ref, o_ref):
    o_ref[...] = (
        jnp.dot(x_ref[...], wt_ref[...], preferred_element_type=jnp.float32)
        + b_ref[...]
    )


def _tc_linear(x, wt, b):
    nblk = 10
    rows = N_NODES // nblk
    return pl.pallas_call(
        _lin_body,
        grid=(nblk,),
        in_specs=[
            pl.BlockSpec((rows, D), lambda i: (i, 0)),
            pl.BlockSpec((D, D), lambda i: (0, 0)),
            pl.BlockSpec((1, D), lambda i: (0, 0)),
        ],
        out_specs=pl.BlockSpec((rows, D), lambda i: (i, 0)),
        out_shape=jax.ShapeDtypeStruct((N_NODES, D), jnp.float32),
    )(x, wt, b)


@functools.lru_cache(maxsize=1)
def _sc_scatter_build():
    mesh = plsc.VectorSubcoreMesh(core_axis_name="c", subcore_axis_name="s")

    @functools.partial(
        pl.kernel,
        mesh=mesh,
        out_type=jax.ShapeDtypeStruct((NC, AGG_ROWS, D), jnp.float32),
        scratch_types=[
            pltpu.VMEM((NCHUNK // 2, C), jnp.int32),
            pltpu.VMEM((NCHUNK // 2, C), jnp.int32),
            pltpu.VMEM((C, D), jnp.float32),
            pltpu.VMEM((C, D), jnp.float32),
            pltpu.VMEM_SHARED((AGG_ROWS, D), jnp.float32),
            pltpu.SemaphoreType.DMA,
            pltpu.SemaphoreType.DMA,
            pltpu.SemaphoreType.DMA,
            pltpu.SemaphoreType.DMA,
        ],
    )
    def sc_scatter(h_hbm, src_hbm, dst_hbm, zrows_hbm, out_hbm,
                   sidx, didx, rows0, rows1, agg, sem0, sem1, sem2, sem3):
        cid = lax.axis_index("c")
        sid = lax.axis_index("s")
        wid = cid * NS + sid

        pltpu.sync_copy(zrows_hbm, rows0)
        for k in range(ZROWS_PER_TILE // C):
            pltpu.sync_copy(rows0, agg.at[pl.ds(sid * ZROWS_PER_TILE + k * C, C)])

        plsc.subcore_barrier()

        def gat(k, buf, sem):
            return pltpu.make_async_copy(h_hbm.at[sidx.at[k]], buf, sem)

        half = NCHUNK // 2
        for hh in range(2):
            pltpu.sync_copy(src_hbm.at[wid, pl.ds(hh * half, half)], sidx)
            pltpu.sync_copy(dst_hbm.at[wid, pl.ds(hh * half, half)], didx)
            gat(0, rows0, sem0).start()

            def sca(k, buf, sem):
                return pltpu.make_async_copy(buf, agg.at[didx.at[k]], sem)

            def body(j, carry):
                a = 2 * j
                b = a + 1
                gat(b, rows1, sem1).start()
                gat(a, rows0, sem0).wait()
                sca(a, rows0, sem2).start(add=True)
                gat(b, rows1, sem1).wait()
                sca(b, rows1, sem3).start(add=True)
                sca(a, rows0, sem2).wait()

                @pl.when(j < half // 2 - 1)
                def _prefetch():
                    gat(a + 2, rows0, sem0).start()

                sca(b, rows1, sem3).wait()
                return carry

            lax.fori_loop(0, half // 2, body, 0)
        plsc.subcore_barrier()

        for k in range(ZROWS_PER_TILE // C):
            r0 = sid * ZROWS_PER_TILE + k * C
            pltpu.sync_copy(agg.at[pl.ds(r0, C)], rows0)
            pltpu.sync_copy(rows0, out_hbm.at[cid, pl.ds(r0, C)])

    return sc_scatter


def _comb_body(p_ref, x_ref, wt_ref, b_ref, t_ref, s_ref, q_ref):
    new = jnp.maximum(p_ref[0] + p_ref[1], 0.0)
    res = jnp.maximum(
        jnp.dot(x_ref[...], wt_ref[...], preferred_element_type=jnp.float32)
        + b_ref[...],
        0.0,
    )
    t = new + res
    t_ref[...] = t
    s_ref[...] = jnp.broadcast_to(jnp.sum(t, axis=0), (1, 8, D))
    q_ref[...] = jnp.broadcast_to(jnp.sum(t * t, axis=0), (1, 8, D))


def _tc_combine(p, x, wt, b):
    nblk = 10
    rows = N_NODES // nblk
    return pl.pallas_call(
        _comb_body,
        grid=(nblk,),
        in_specs=[
            pl.BlockSpec((NC, rows, D), lambda i: (0, i, 0)),
            pl.BlockSpec((rows, D), lambda i: (i, 0)),
            pl.BlockSpec((D, D), lambda i: (0, 0)),
            pl.BlockSpec((1, D), lambda i: (0, 0)),
        ],
        out_specs=[
            pl.BlockSpec((rows, D), lambda i: (i, 0)),
            pl.BlockSpec((1, 8, D), lambda i: (i, 0, 0)),
            pl.BlockSpec((1, 8, D), lambda i: (i, 0, 0)),
        ],
        out_shape=[
            jax.ShapeDtypeStruct((N_NODES, D), jnp.float32),
            jax.ShapeDtypeStruct((nblk, 8, D), jnp.float32),
            jax.ShapeDtypeStruct((nblk, 8, D), jnp.float32),
        ],
    )(p, x, wt, b)


def _norm_body(t_ref, s_ref, q_ref, g_ref, bt_ref, o_ref):
    n = float(N_NODES)
    mean = jnp.sum(s_ref[:, 0, :], axis=0, keepdims=True) / n
    var = jnp.sum(q_ref[:, 0, :], axis=0, keepdims=True) / n - mean * mean
    inv = lax.rsqrt(var + EPS)
    o_ref[...] = (t_ref[...] - mean) * (inv * g_ref[...]) + bt_ref[...]


def _tc_norm(t, s, q, gamma, beta):
    nblk = 10
    rows = N_NODES // nblk
    return pl.pallas_call(
        _norm_body,
        grid=(nblk,),
        in_specs=[
            pl.BlockSpec((rows, D), lambda i: (i, 0)),
            pl.BlockSpec((nblk, 8, D), lambda i: (0, 0, 0)),
            pl.BlockSpec((nblk, 8, D), lambda i: (0, 0, 0)),
            pl.BlockSpec((1, D), lambda i: (0, 0)),
            pl.BlockSpec((1, D), lambda i: (0, 0)),
        ],
        out_specs=pl.BlockSpec((rows, D), lambda i: (i, 0)),
        out_shape=jax.ShapeDtypeStruct((N_NODES, D), jnp.float32),
    )(t, s, q, gamma, beta)


def kernel(feats, edge_index, W_rel, b_rel, W_res, b_res, gamma, beta):
    src = edge_index[0].astype(jnp.int32)
    dst = edge_index[1].astype(jnp.int32)
    pad = E_PAD - src.shape[0]
    src = jnp.concatenate([src, jnp.zeros((pad,), jnp.int32)])
    dst = jnp.concatenate([dst, jnp.full((pad,), DUMMY_DST, jnp.int32)])
    src3 = src.reshape(NW, NCHUNK, C)
    dst3 = dst.reshape(NW, NCHUNK, C)
    zrows = jnp.zeros((C, D), jnp.float32)

    h = _tc_linear(feats, W_rel.T, b_rel.reshape(1, D))
    p = _sc_scatter_build()(h, src3, dst3, zrows)
    t, s, q = _tc_combine(p, feats, W_res.T, b_res.reshape(1, D))
    return _tc_norm(t, s, q, gamma.reshape(1, D), beta.reshape(1, D))


# speedup vs baseline: 2.5791x; 2.5791x over previous
import functools

import jax
import jax.numpy as jnp
from jax import lax
from jax.experimental import pallas as pl
from jax.experimental.pallas import tpu as pltpu
from jax.experimental.pallas import tpu_sc as plsc

N_NODES = 10000
D = 128
EPS = 1e-5

NC = 2
NS = 16
NW = NC * NS
C = 128
NCHUNK = 80
EPT = C * NCHUNK
E_PAD = NW * EPT
AGG_ROWS = 10240
DUMMY_DST = 10100
ZROWS_PER_TILE = AGG_ROWS // NS
OROWS_PER_TILE = N_NODES // NS


def _lin_body(x_ref, wt_ref, b_ref, o_ref):
    o_ref[...] = (
        jnp.dot(x_ref[...], wt_ref[...], preferred_element_type=jnp.float32)
        + b_ref[...]
    )


def _tc_linear(x, wt, b):
    nblk = 10
    rows = N_NODES // nblk
    return pl.pallas_call(
        _lin_body,
        grid=(nblk,),
        in_specs=[
            pl.BlockSpec((rows, D), lambda i: (i, 0)),
            pl.BlockSpec((D, D), lambda i: (0, 0)),
            pl.BlockSpec((1, D), lambda i: (0, 0)),
        ],
        out_specs=pl.BlockSpec((rows, D), lambda i: (i, 0)),
        out_shape=jax.ShapeDtypeStruct((N_NODES, D), jnp.float32),
    )(x, wt, b)


@functools.lru_cache(maxsize=1)
def _sc_scatter_build():
    mesh = plsc.VectorSubcoreMesh(core_axis_name="c", subcore_axis_name="s")

    @functools.partial(
        pl.kernel,
        mesh=mesh,
        out_type=jax.ShapeDtypeStruct((NC, AGG_ROWS, D), jnp.float32),
        scratch_types=[
            pltpu.VMEM((NCHUNK // 2, C), jnp.int32),
            pltpu.VMEM((NCHUNK // 2, C), jnp.int32),
            pltpu.VMEM((C, D), jnp.float32),
            pltpu.VMEM((C, D), jnp.float32),
            pltpu.VMEM_SHARED((AGG_ROWS, D), jnp.float32),
            pltpu.SemaphoreType.DMA,
            pltpu.SemaphoreType.DMA,
            pltpu.SemaphoreType.DMA,
            pltpu.SemaphoreType.DMA,
        ],
    )
    def sc_scatter(h_hbm, src_hbm, dst_hbm, zrows_hbm, out_hbm,
                   sidx, didx, rows0, rows1, agg, sem0, sem1, sem2, sem3):
        cid = lax.axis_index("c")
        sid = lax.axis_index("s")
        wid = cid * NS + sid

        pltpu.sync_copy(zrows_hbm, rows0)
        for k in range(ZROWS_PER_TILE // C):
            pltpu.sync_copy(rows0, agg.at[pl.ds(sid * ZROWS_PER_TILE + k * C, C)])

        plsc.subcore_barrier()

        def gat(k, buf, sem):
            return pltpu.make_async_copy(h_hbm.at[sidx.at[k]], buf, sem)

        half = NCHUNK // 2
        for hh in range(2):
            pltpu.sync_copy(src_hbm.at[wid, pl.ds(hh * half, half)], sidx)
            pltpu.sync_copy(dst_hbm.at[wid, pl.ds(hh * half, half)], didx)
            gat(0, rows0, sem0).start()

            def sca(k, buf, sem):
                return pltpu.make_async_copy(buf, agg.at[didx.at[k]], sem)

            def body(j, carry):
                a = 2 * j
                b = a + 1
                gat(b, rows1, sem1).start()
                gat(a, rows0, sem0).wait()
                sca(a, rows0, sem2).start(add=True)
                gat(b, rows1, sem1).wait()
                sca(b, rows1, sem3).start(add=True)
                sca(a, rows0, sem2).wait()

                @pl.when(j < half // 2 - 1)
                def _prefetch():
                    gat(a + 2, rows0, sem0).start()

                sca(b, rows1, sem3).wait()
                return carry

            lax.fori_loop(0, half // 2, body, 0)
        plsc.subcore_barrier()

        for k in range(ZROWS_PER_TILE // C):
            r0 = sid * ZROWS_PER_TILE + k * C
            pltpu.sync_copy(agg.at[pl.ds(r0, C)], rows0)
            pltpu.sync_copy(rows0, out_hbm.at[cid, pl.ds(r0, C)])

    return sc_scatter


def _comb_body(p_ref, x_ref, wt_ref, b_ref, t_ref, s_ref, q_ref):
    new = jnp.maximum(p_ref[0] + p_ref[1], 0.0)
    res = jnp.maximum(
        jnp.dot(x_ref[...], wt_ref[...], preferred_element_type=jnp.float32)
        + b_ref[...],
        0.0,
    )
    t = new + res
    t_ref[...] = t
    s_ref[...] = jnp.broadcast_to(jnp.sum(t, axis=0), (1, 8, D))
    q_ref[...] = jnp.broadcast_to(jnp.sum(t * t, axis=0), (1, 8, D))


def _tc_combine(p, x, wt, b):
    nblk = 10
    rows = N_NODES // nblk
    return pl.pallas_call(
        _comb_body,
        grid=(nblk,),
        in_specs=[
            pl.BlockSpec((NC, rows, D), lambda i: (0, i, 0)),
            pl.BlockSpec((rows, D), lambda i: (i, 0)),
            pl.BlockSpec((D, D), lambda i: (0, 0)),
            pl.BlockSpec((1, D), lambda i: (0, 0)),
        ],
        out_specs=[
            pl.BlockSpec((rows, D), lambda i: (i, 0)),
            pl.BlockSpec((1, 8, D), lambda i: (i, 0, 0)),
            pl.BlockSpec((1, 8, D), lambda i: (i, 0, 0)),
        ],
        out_shape=[
            jax.ShapeDtypeStruct((N_NODES, D), jnp.float32),
            jax.ShapeDtypeStruct((nblk, 8, D), jnp.float32),
            jax.ShapeDtypeStruct((nblk, 8, D), jnp.float32),
        ],
    )(p, x, wt, b)


def _norm_body(t_ref, s_ref, q_ref, g_ref, bt_ref, o_ref):
    n = float(N_NODES)
    mean = jnp.sum(s_ref[:, 0, :], axis=0, keepdims=True) / n
    var = jnp.sum(q_ref[:, 0, :], axis=0, keepdims=True) / n - mean * mean
    inv = lax.rsqrt(var + EPS)
    o_ref[...] = (t_ref[...] - mean) * (inv * g_ref[...]) + bt_ref[...]


def _tc_norm(t, s, q, gamma, beta):
    nblk = 10
    rows = N_NODES // nblk
    return pl.pallas_call(
        _norm_body,
        grid=(nblk,),
        in_specs=[
            pl.BlockSpec((rows, D), lambda i: (i, 0)),
            pl.BlockSpec((nblk, 8, D), lambda i: (0, 0, 0)),
            pl.BlockSpec((nblk, 8, D), lambda i: (0, 0, 0)),
            pl.BlockSpec((1, D), lambda i: (0, 0)),
            pl.BlockSpec((1, D), lambda i: (0, 0)),
        ],
        out_specs=pl.BlockSpec((rows, D), lambda i: (i, 0)),
        out_shape=jax.ShapeDtypeStruct((N_NODES, D), jnp.float32),
    )(t, s, q, gamma, beta)


def kernel(feats, edge_index, W_rel, b_rel, W_res, b_res, gamma, beta):
    src = edge_index[0].astype(jnp.int32)
    dst = edge_index[1].astype(jnp.int32)
    pad = E_PAD - src.shape[0]
    pad_iota = jnp.arange(pad, dtype=jnp.int32)
    src = jnp.concatenate([src, pad_iota % N_NODES])
    dst = jnp.concatenate([dst, N_NODES + pad_iota % (AGG_ROWS - N_NODES)])
    src3 = src.reshape(NW, NCHUNK, C)
    dst3 = dst.reshape(NW, NCHUNK, C)
    zrows = jnp.zeros((C, D), jnp.float32)

    h = _tc_linear(feats, W_rel.T, b_rel.reshape(1, D))
    p = _sc_scatter_build()(h, src3, dst3, zrows)
    t, s, q = _tc_combine(p, feats, W_res.T, b_res.reshape(1, D))
    return _tc_norm(t, s, q, gamma.reshape(1, D), beta.reshape(1, D))


# speedup vs baseline: 2.8864x; 1.1192x over previous
import functools

import jax
import jax.numpy as jnp
from jax import lax
from jax.experimental import pallas as pl
from jax.experimental.pallas import tpu as pltpu
from jax.experimental.pallas import tpu_sc as plsc

N_NODES = 10000
D = 128
EPS = 1e-5

NC = 2
NS = 16
NW = NC * NS
C = 64
NCHUNK = 160
DEPTH = 4
EPT = C * NCHUNK
E_PAD = NW * EPT
AGG_ROWS = 10240
ZROWS_PER_TILE = AGG_ROWS // NS


def _lin_body(x_ref, wt_ref, b_ref, o_ref):
    o_ref[...] = (
        jnp.dot(x_ref[...], wt_ref[...], preferred_element_type=jnp.float32)
        + b_ref[...]
    )


def _tc_linear(x, wt, b):
    nblk = 10
    rows = N_NODES // nblk
    return pl.pallas_call(
        _lin_body,
        grid=(nblk,),
        in_specs=[
            pl.BlockSpec((rows, D), lambda i: (i, 0)),
            pl.BlockSpec((D, D), lambda i: (0, 0)),
            pl.BlockSpec((1, D), lambda i: (0, 0)),
        ],
        out_specs=pl.BlockSpec((rows, D), lambda i: (i, 0)),
        out_shape=jax.ShapeDtypeStruct((N_NODES, D), jnp.float32),
    )(x, wt, b)


@functools.lru_cache(maxsize=1)
def _sc_scatter_build():
    mesh = plsc.VectorSubcoreMesh(core_axis_name="c", subcore_axis_name="s")

    scratch = (
        [pltpu.VMEM((NCHUNK // 4, C), jnp.int32)] * 2
        + [pltpu.VMEM((C, D), jnp.float32)] * DEPTH
        + [pltpu.VMEM_SHARED((AGG_ROWS, D), jnp.float32)]
        + [pltpu.SemaphoreType.DMA] * (2 * DEPTH)
    )

    @functools.partial(
        pl.kernel,
        mesh=mesh,
        out_type=jax.ShapeDtypeStruct((NC, AGG_ROWS, D), jnp.float32),
        scratch_types=scratch,
    )
    def sc_scatter(h_hbm, src_hbm, dst_hbm, zrows_hbm, out_hbm,
                   sidx, didx, *rest):
        bufs = rest[:DEPTH]
        agg = rest[DEPTH]
        gsem = rest[DEPTH + 1:DEPTH + 1 + DEPTH]
        ssem = rest[DEPTH + 1 + DEPTH:]
        cid = lax.axis_index("c")
        sid = lax.axis_index("s")
        wid = cid * NS + sid

        pltpu.sync_copy(zrows_hbm, bufs[0])
        for k in range(ZROWS_PER_TILE // C):
            pltpu.sync_copy(bufs[0],
                            agg.at[pl.ds(sid * ZROWS_PER_TILE + k * C, C)])

        plsc.subcore_barrier()

        def gat(k, b):
            return pltpu.make_async_copy(h_hbm.at[sidx.at[k]], bufs[b], gsem[b])

        def sca(k, b):
            return pltpu.make_async_copy(bufs[b], agg.at[didx.at[k]], ssem[b])

        half = NCHUNK // 4
        nit = half // DEPTH
        for hh in range(4):
            pltpu.sync_copy(src_hbm.at[wid, pl.ds(hh * half, half)], sidx)
            pltpu.sync_copy(dst_hbm.at[wid, pl.ds(hh * half, half)], didx)
            for b in range(DEPTH):
                gat(b, b).start()

            def body(j, carry):
                for b in range(DEPTH):
                    k = DEPTH * j + b
                    gat(k, b).wait()
                    sca(k, b).start(add=True)
                for b in range(DEPTH):
                    k = DEPTH * j + b
                    sca(k, b).wait()

                    @pl.when(j < nit - 1)
                    def _prefetch():
                        gat(k + DEPTH, b).start()

                return carry

            lax.fori_loop(0, nit, body, 0)
        plsc.subcore_barrier()

        for k in range(ZROWS_PER_TILE // C):
            r0 = sid * ZROWS_PER_TILE + k * C
            pltpu.sync_copy(agg.at[pl.ds(r0, C)], bufs[0])
            pltpu.sync_copy(bufs[0], out_hbm.at[cid, pl.ds(r0, C)])

    return sc_scatter


def _comb_body(p_ref, x_ref, wt_ref, b_ref, t_ref, s_ref, q_ref):
    new = jnp.maximum(p_ref[0] + p_ref[1], 0.0)
    res = jnp.maximum(
        jnp.dot(x_ref[...], wt_ref[...], preferred_element_type=jnp.float32)
        + b_ref[...],
        0.0,
    )
    t = new + res
    t_ref[...] = t
    s_ref[...] = jnp.broadcast_to(jnp.sum(t, axis=0), (1, 8, D))
    q_ref[...] = jnp.broadcast_to(jnp.sum(t * t, axis=0), (1, 8, D))


def _tc_combine(p, x, wt, b):
    nblk = 10
    rows = N_NODES // nblk
    return pl.pallas_call(
        _comb_body,
        grid=(nblk,),
        in_specs=[
            pl.BlockSpec((NC, rows, D), lambda i: (0, i, 0)),
            pl.BlockSpec((rows, D), lambda i: (i, 0)),
            pl.BlockSpec((D, D), lambda i: (0, 0)),
            pl.BlockSpec((1, D), lambda i: (0, 0)),
        ],
        out_specs=[
            pl.BlockSpec((rows, D), lambda i: (i, 0)),
            pl.BlockSpec((1, 8, D), lambda i: (i, 0, 0)),
            pl.BlockSpec((1, 8, D), lambda i: (i, 0, 0)),
        ],
        out_shape=[
            jax.ShapeDtypeStruct((N_NODES, D), jnp.float32),
            jax.ShapeDtypeStruct((nblk, 8, D), jnp.float32),
            jax.ShapeDtypeStruct((nblk, 8, D), jnp.float32),
        ],
    )(p, x, wt, b)


def _norm_body(t_ref, s_ref, q_ref, g_ref, bt_ref, o_ref):
    n = float(N_NODES)
    mean = jnp.sum(s_ref[:, 0, :], axis=0, keepdims=True) / n
    var = jnp.sum(q_ref[:, 0, :], axis=0, keepdims=True) / n - mean * mean
    inv = lax.rsqrt(var + EPS)
    o_ref[...] = (t_ref[...] - mean) * (inv * g_ref[...]) + bt_ref[...]


def _tc_norm(t, s, q, gamma, beta):
    nblk = 10
    rows = N_NODES // nblk
    return pl.pallas_call(
        _norm_body,
        grid=(nblk,),
        in_specs=[
            pl.BlockSpec((rows, D), lambda i: (i, 0)),
            pl.BlockSpec((nblk, 8, D), lambda i: (0, 0, 0)),
            pl.BlockSpec((nblk, 8, D), lambda i: (0, 0, 0)),
            pl.BlockSpec((1, D), lambda i: (0, 0)),
            pl.BlockSpec((1, D), lambda i: (0, 0)),
        ],
        out_specs=pl.BlockSpec((rows, D), lambda i: (i, 0)),
        out_shape=jax.ShapeDtypeStruct((N_NODES, D), jnp.float32),
    )(t, s, q, gamma, beta)


def kernel(feats, edge_index, W_rel, b_rel, W_res, b_res, gamma, beta):
    src = edge_index[0].astype(jnp.int32)
    dst = edge_index[1].astype(jnp.int32)
    pad = E_PAD - src.shape[0]
    pad_iota = jnp.arange(pad, dtype=jnp.int32)
    src = jnp.concatenate([src, pad_iota % N_NODES])
    dst = jnp.concatenate([dst, N_NODES + pad_iota % (AGG_ROWS - N_NODES)])
    src3 = src.reshape(NW, NCHUNK, C)
    dst3 = dst.reshape(NW, NCHUNK, C)
    zrows = jnp.zeros((C, D), jnp.float32)

    h = _tc_linear(feats, W_rel.T, b_rel.reshape(1, D))
    p = _sc_scatter_build()(h, src3, dst3, zrows)
    t, s, q = _tc_combine(p, feats, W_res.T, b_res.reshape(1, D))
    return _tc_norm(t, s, q, gamma.reshape(1, D), beta.reshape(1, D))


# speedup vs baseline: 2.9853x; 1.0343x over previous
import functools

import jax
import jax.numpy as jnp
from jax import lax
from jax.experimental import pallas as pl
from jax.experimental.pallas import tpu as pltpu
from jax.experimental.pallas import tpu_sc as plsc

N_NODES = 10000
D = 128
EPS = 1e-5

NC = 2
NS = 16
NW = NC * NS
C = 64
NCHUNK = 160
DEPTH = 4
EPT = C * NCHUNK
E_PAD = NW * EPT
AGG_ROWS = 10240
ZROWS_PER_TILE = AGG_ROWS // NS


def _matT(x, w):
    return lax.dot_general(x, w, (((1,), (1,)), ((), ())),
                           preferred_element_type=jnp.float32)


def _lin_body(x_ref, w_ref, b_ref, o_ref):
    o_ref[...] = _matT(x_ref[...], w_ref[...]) + b_ref[...][None, :]


def _tc_linear(x, w, b):
    nblk = 10
    rows = N_NODES // nblk
    return pl.pallas_call(
        _lin_body,
        grid=(nblk,),
        in_specs=[
            pl.BlockSpec((rows, D), lambda i: (i, 0)),
            pl.BlockSpec((D, D), lambda i: (0, 0)),
            pl.BlockSpec((D,), lambda i: (0,)),
        ],
        out_specs=pl.BlockSpec((rows, D), lambda i: (i, 0)),
        out_shape=jax.ShapeDtypeStruct((N_NODES, D), jnp.float32),
    )(x, w, b)


@functools.lru_cache(maxsize=1)
def _sc_scatter_build():
    mesh = plsc.VectorSubcoreMesh(core_axis_name="c", subcore_axis_name="s")

    scratch = (
        [pltpu.VMEM((NCHUNK // 4, C), jnp.int32)] * 2
        + [pltpu.VMEM((C, D), jnp.float32)] * DEPTH
        + [pltpu.VMEM_SHARED((AGG_ROWS, D), jnp.float32)]
        + [pltpu.SemaphoreType.DMA] * (2 * DEPTH)
    )

    @functools.partial(
        pl.kernel,
        mesh=mesh,
        out_type=jax.ShapeDtypeStruct((NC, AGG_ROWS, D), jnp.float32),
        scratch_types=scratch,
    )
    def sc_scatter(h_hbm, src_hbm, dst_hbm, zrows_hbm, out_hbm,
                   sidx, didx, *rest):
        bufs = rest[:DEPTH]
        agg = rest[DEPTH]
        gsem = rest[DEPTH + 1:DEPTH + 1 + DEPTH]
        ssem = rest[DEPTH + 1 + DEPTH:]
        cid = lax.axis_index("c")
        sid = lax.axis_index("s")
        wid = cid * NS + sid

        pltpu.sync_copy(zrows_hbm, bufs[0])
        for k in range(ZROWS_PER_TILE // C):
            pltpu.sync_copy(bufs[0],
                            agg.at[pl.ds(sid * ZROWS_PER_TILE + k * C, C)])

        plsc.subcore_barrier()

        def gat(k, b):
            return pltpu.make_async_copy(h_hbm.at[sidx.at[k]], bufs[b], gsem[b])

        def sca(k, b):
            return pltpu.make_async_copy(bufs[b], agg.at[didx.at[k]], ssem[b])

        half = NCHUNK // 4
        nit = half // DEPTH
        for hh in range(4):
            pltpu.sync_copy(src_hbm.at[wid, pl.ds(hh * half, half)], sidx)
            pltpu.sync_copy(dst_hbm.at[wid, pl.ds(hh * half, half)], didx)
            for b in range(DEPTH):
                gat(b, b).start()

            def body(j, carry):
                for b in range(DEPTH):
                    k = DEPTH * j + b
                    gat(k, b).wait()
                    sca(k, b).start(add=True)
                for b in range(DEPTH):
                    k = DEPTH * j + b
                    sca(k, b).wait()

                    @pl.when(j < nit - 1)
                    def _prefetch():
                        gat(k + DEPTH, b).start()

                return carry

            lax.fori_loop(0, nit, body, 0)
        plsc.subcore_barrier()

        for k in range(ZROWS_PER_TILE // C):
            r0 = sid * ZROWS_PER_TILE + k * C
            pltpu.sync_copy(agg.at[pl.ds(r0, C)], bufs[0])
            pltpu.sync_copy(bufs[0], out_hbm.at[cid, pl.ds(r0, C)])

    return sc_scatter


_NBLK = 10
_ROWS = N_NODES // _NBLK


def _comb_body(p_ref, x_ref, w_ref, b_ref, g_ref, bt_ref, o_ref,
               t_ref, s_ref, q_ref):
    ph = pl.program_id(0)
    i = pl.program_id(1)

    @pl.when(ph == 0)
    def _compute():
        new = jnp.maximum(p_ref[0] + p_ref[1], 0.0)
        res = jnp.maximum(_matT(x_ref[...], w_ref[...]) + b_ref[...][None, :],
                          0.0)
        t = new + res
        t_ref[pl.ds(i * _ROWS, _ROWS), :] = t
        s_ref[pl.ds(i, 1), :] = jnp.sum(t, axis=0, keepdims=True)
        q_ref[pl.ds(i, 1), :] = jnp.sum(t * t, axis=0, keepdims=True)

    @pl.when(ph == 1)
    def _normalize():
        n = float(N_NODES)
        mean = jnp.sum(s_ref[...], axis=0, keepdims=True) / n
        var = jnp.sum(q_ref[...], axis=0, keepdims=True) / n - mean * mean
        inv = lax.rsqrt(var + EPS)
        t = t_ref[pl.ds(i * _ROWS, _ROWS), :]
        o_ref[...] = (t - mean) * (inv * g_ref[...][None, :]) + bt_ref[...][None, :]


def _tc_combine(p, x, w, b, gamma, beta):
    return pl.pallas_call(
        _comb_body,
        grid=(2, _NBLK),
        in_specs=[
            pl.BlockSpec((NC, _ROWS, D), lambda p_, i: (0, (1 - p_) * i, 0)),
            pl.BlockSpec((_ROWS, D), lambda p_, i: ((1 - p_) * i, 0)),
            pl.BlockSpec((D, D), lambda p_, i: (0, 0)),
            pl.BlockSpec((D,), lambda p_, i: (0,)),
            pl.BlockSpec((D,), lambda p_, i: (0,)),
            pl.BlockSpec((D,), lambda p_, i: (0,)),
        ],
        out_specs=pl.BlockSpec((_ROWS, D), lambda p_, i: (p_ * i, 0)),
        out_shape=jax.ShapeDtypeStruct((N_NODES, D), jnp.float32),
        scratch_shapes=[
            pltpu.VMEM((N_NODES, D), jnp.float32),
            pltpu.VMEM((_NBLK, D), jnp.float32),
            pltpu.VMEM((_NBLK, D), jnp.float32),
        ],
    )(p, x, w, b, gamma, beta)


def kernel(feats, edge_index, W_rel, b_rel, W_res, b_res, gamma, beta):
    src = edge_index[0].astype(jnp.int32)
    dst = edge_index[1].astype(jnp.int32)
    pad = E_PAD - src.shape[0]
    pad_iota = jnp.arange(pad, dtype=jnp.int32)
    src = jnp.concatenate([src, pad_iota % N_NODES])
    dst = jnp.concatenate([dst, N_NODES + pad_iota % (AGG_ROWS - N_NODES)])
    src3 = src.reshape(NW, NCHUNK, C)
    dst3 = dst.reshape(NW, NCHUNK, C)
    zrows = jnp.zeros((C, D), jnp.float32)

    h = _tc_linear(feats, W_rel, b_rel)
    p = _sc_scatter_build()(h, src3, dst3, zrows)
    return _tc_combine(p, feats, W_res, b_res, gamma, beta)


# speedup vs baseline: 3.0018x; 1.0055x over previous
import functools

import jax
import jax.numpy as jnp
from jax import lax
from jax.experimental import pallas as pl
from jax.experimental.pallas import tpu as pltpu
from jax.experimental.pallas import tpu_sc as plsc

N_NODES = 10000
D = 128
EPS = 1e-5

NC = 2
NS = 16
NW = NC * NS
C = 64
NCHUNK = 160
DEPTH = 4
EPT = C * NCHUNK
E_PAD = NW * EPT
AGG_ROWS = 10240
ZROWS_PER_TILE = AGG_ROWS // NS


def _matT(x, w):
    return lax.dot_general(x, w, (((1,), (1,)), ((), ())),
                           preferred_element_type=jnp.float32)


def _lin_body(x_ref, w_ref, b_ref, o_ref):
    o_ref[...] = _matT(x_ref[...], w_ref[...]) + b_ref[...][None, :]


def _tc_linear(x, w, b):
    nblk = 10
    rows = N_NODES // nblk
    return pl.pallas_call(
        _lin_body,
        grid=(nblk,),
        in_specs=[
            pl.BlockSpec((rows, D), lambda i: (i, 0)),
            pl.BlockSpec((D, D), lambda i: (0, 0)),
            pl.BlockSpec((D,), lambda i: (0,)),
        ],
        out_specs=pl.BlockSpec((rows, D), lambda i: (i, 0)),
        out_shape=jax.ShapeDtypeStruct((N_NODES, D), jnp.float32),
    )(x, w, b)


@functools.lru_cache(maxsize=1)
def _sc_scatter_build():
    mesh = plsc.VectorSubcoreMesh(core_axis_name="c", subcore_axis_name="s")

    scratch = (
        [pltpu.VMEM((NCHUNK // 4, C), jnp.int32)] * 2
        + [pltpu.VMEM((C, D), jnp.float32)] * DEPTH
        + [pltpu.VMEM_SHARED((AGG_ROWS, D), jnp.float32)]
        + [pltpu.SemaphoreType.DMA] * (2 * DEPTH)
    )

    @functools.partial(
        pl.kernel,
        mesh=mesh,
        out_type=jax.ShapeDtypeStruct((NC, AGG_ROWS, D), jnp.float32),
        scratch_types=scratch,
    )
    def sc_scatter(h_hbm, src_hbm, dst_hbm, zrows_hbm, out_hbm,
                   sidx, didx, *rest):
        bufs = rest[:DEPTH]
        agg = rest[DEPTH]
        gsem = rest[DEPTH + 1:DEPTH + 1 + DEPTH]
        ssem = rest[DEPTH + 1 + DEPTH:]
        cid = lax.axis_index("c")
        sid = lax.axis_index("s")
        wid = cid * NS + sid

        pltpu.sync_copy(zrows_hbm, bufs[0])
        for k in range(ZROWS_PER_TILE // C):
            pltpu.sync_copy(bufs[0],
                            agg.at[pl.ds(sid * ZROWS_PER_TILE + k * C, C)])

        plsc.subcore_barrier()

        def gat(k, b):
            return pltpu.make_async_copy(h_hbm.at[sidx.at[k]], bufs[b], gsem[b])

        def sca(k, b):
            return pltpu.make_async_copy(bufs[b], agg.at[didx.at[k]], ssem[b])

        half = NCHUNK // 4
        nit = half // DEPTH
        for hh in range(4):
            pltpu.sync_copy(src_hbm.at[wid, pl.ds(hh * half, half)], sidx)
            pltpu.sync_copy(dst_hbm.at[wid, pl.ds(hh * half, half)], didx)
            for b in range(DEPTH):
                gat(b, b).start()

            def body(j, carry):
                for b in range(DEPTH):
                    k = DEPTH * j + b
                    gat(k, b).wait()
                    sca(k, b).start(add=True)
                for b in range(DEPTH):
                    k = DEPTH * j + b
                    sca(k, b).wait()

                    @pl.when(j < nit - 1)
                    def _prefetch():
                        gat(k + DEPTH, b).start()

                return carry

            lax.fori_loop(0, nit, body, 0)
        plsc.subcore_barrier()

        r0 = sid * ZROWS_PER_TILE
        pltpu.sync_copy(agg.at[pl.ds(r0, ZROWS_PER_TILE)],
                        out_hbm.at[cid, pl.ds(r0, ZROWS_PER_TILE)])

    return sc_scatter


_NBLK = 10
_ROWS = N_NODES // _NBLK


def _comb_body(p_ref, x_ref, w_ref, b_ref, g_ref, bt_ref, o_ref,
               t_ref, s_ref, q_ref):
    ph = pl.program_id(0)
    i = pl.program_id(1)

    @pl.when(ph == 0)
    def _compute():
        new = jnp.maximum(p_ref[0] + p_ref[1], 0.0)
        res = jnp.maximum(_matT(x_ref[...], w_ref[...]) + b_ref[...][None, :],
                          0.0)
        t = new + res
        t_ref[pl.ds(i * _ROWS, _ROWS), :] = t
        s_ref[pl.ds(i, 1), :] = jnp.sum(t, axis=0, keepdims=True)
        q_ref[pl.ds(i, 1), :] = jnp.sum(t * t, axis=0, keepdims=True)

    @pl.when(ph == 1)
    def _normalize():
        n = float(N_NODES)
        mean = jnp.sum(s_ref[...], axis=0, keepdims=True) / n
        var = jnp.sum(q_ref[...], axis=0, keepdims=True) / n - mean * mean
        inv = lax.rsqrt(var + EPS)
        t = t_ref[pl.ds(i * _ROWS, _ROWS), :]
        o_ref[...] = (t - mean) * (inv * g_ref[...][None, :]) + bt_ref[...][None, :]


def _tc_combine(p, x, w, b, gamma, beta):
    return pl.pallas_call(
        _comb_body,
        grid=(2, _NBLK),
        in_specs=[
            pl.BlockSpec((NC, _ROWS, D), lambda p_, i: (0, (1 - p_) * i, 0)),
            pl.BlockSpec((_ROWS, D), lambda p_, i: ((1 - p_) * i, 0)),
            pl.BlockSpec((D, D), lambda p_, i: (0, 0)),
            pl.BlockSpec((D,), lambda p_, i: (0,)),
            pl.BlockSpec((D,), lambda p_, i: (0,)),
            pl.BlockSpec((D,), lambda p_, i: (0,)),
        ],
        out_specs=pl.BlockSpec((_ROWS, D), lambda p_, i: (p_ * i, 0)),
        out_shape=jax.ShapeDtypeStruct((N_NODES, D), jnp.float32),
        scratch_shapes=[
            pltpu.VMEM((N_NODES, D), jnp.float32),
            pltpu.VMEM((_NBLK, D), jnp.float32),
            pltpu.VMEM((_NBLK, D), jnp.float32),
        ],
    )(p, x, w, b, gamma, beta)


def kernel(feats, edge_index, W_rel, b_rel, W_res, b_res, gamma, beta):
    src = edge_index[0].astype(jnp.int32)
    dst = edge_index[1].astype(jnp.int32)
    pad = E_PAD - src.shape[0]
    pad_iota = jnp.arange(pad, dtype=jnp.int32)
    src = jnp.concatenate([src, pad_iota % N_NODES])
    dst = jnp.concatenate([dst, N_NODES + pad_iota % (AGG_ROWS - N_NODES)])
    src3 = src.reshape(NW, NCHUNK, C)
    dst3 = dst.reshape(NW, NCHUNK, C)
    zrows = jnp.zeros((C, D), jnp.float32)

    h = _tc_linear(feats, W_rel, b_rel)
    p = _sc_scatter_build()(h, src3, dst3, zrows)
    return _tc_combine(p, feats, W_res, b_res, gamma, beta)
